# baseline (speedup 1.0000x reference)
import functools

import jax
import jax.numpy as jnp
from jax import lax
from jax.experimental import pallas as pl
from jax.experimental.pallas import tpu as pltpu
from jax.experimental.pallas import tpu_sc as plsc

D_MODEL = 1024
D_EXPERT = 2048
N_ROUTED = 8
TOP_K = 2
M_G = 128
M_TILE = 512

_NT = (((1,), (1,)), ((), ()))


def _shared_kernel(x_ref, wug_ref, wd_ref, out_ref):
    xb = x_ref[...].astype(jnp.bfloat16)
    ug = lax.dot_general(xb, wug_ref[...], _NT, preferred_element_type=jnp.float32)
    u = ug[:, :D_EXPERT]
    g = ug[:, D_EXPERT:]
    a = (u * lax.logistic(u) * g).astype(jnp.bfloat16)
    out_ref[...] = lax.dot_general(a, wd_ref[...], _NT,
                                   preferred_element_type=jnp.float32)


def _build_kernel(x_ref, rw_ref, pos0_ref, pos1_ref, w2_ref, te_ref, tot_ref, *,
                  n_tokens, n_tiles):
    lg = lax.dot_general(x_ref[...], rw_ref[...], _NT,
                         preferred_element_type=jnp.float32)
    mx = jnp.max(lg, axis=-1, keepdims=True)
    p = jnp.exp(lg - mx)
    p = p / jnp.sum(p, axis=-1, keepdims=True)
    cols = lax.broadcasted_iota(jnp.int32, p.shape, 1)
    m1 = jnp.max(p, axis=-1, keepdims=True)
    i1 = jnp.min(jnp.where(p == m1, cols, N_ROUTED), axis=-1, keepdims=True)
    sel1 = cols == i1
    pm = jnp.where(sel1, -jnp.inf, p)
    m2 = jnp.max(pm, axis=-1, keepdims=True)
    i2 = jnp.min(jnp.where(pm == m2, cols, N_ROUTED), axis=-1, keepdims=True)
    den = m1 + m2 + 1e-8
    w2_ref[...] = jnp.concatenate([m1 / den, m2 / den], axis=1)

    key = jnp.concatenate([i1, i2], axis=0)
    oh = (key == lax.broadcasted_iota(jnp.int32, (2 * n_tokens, N_ROUTED), 1)
          ).astype(jnp.float32)
    inc = oh
    s = 1
    while s < 2 * n_tokens:
        inc = inc + jnp.concatenate(
            [jnp.zeros((s, N_ROUTED), jnp.float32), inc[:-s, :]], axis=0)
        s *= 2
    counts = inc[2 * n_tokens - 1:, :]
    cnt_pad = jnp.floor((counts + (M_G - 1)) * (1.0 / M_G)) * M_G
    incl = cnt_pad
    s = 1
    while s < N_ROUTED:
        incl = incl + jnp.concatenate(
            [jnp.zeros((1, s), jnp.float32), incl[:, :-s]], axis=1)
        s *= 2
    offs = incl - cnt_pad
    rank = jnp.sum(inc * oh, axis=1, keepdims=True) - 1.0
    offsel = jnp.sum(jnp.where(oh > 0.0, offs, 0.0), axis=1, keepdims=True)
    pos = (offsel + rank).astype(jnp.int32)
    pos0_ref[...] = pos[:n_tokens]
    pos1_ref[...] = pos[n_tokens:]

    t_row = (lax.broadcasted_iota(jnp.int32, (n_tiles, N_ROUTED), 0)
             * M_G).astype(jnp.float32)
    te = jnp.sum((offs <= t_row).astype(jnp.int32), axis=1, keepdims=True) - 1
    te_ref[...] = te
    tot_ref[...] = jnp.sum(cnt_pad, axis=1, keepdims=True).astype(jnp.int32)


def _sc_scatter_rows(x_i32, pos0, pos1, r_cap):
    n, c2 = x_i32.shape
    info = plsc.get_sparse_core_info()
    nw = info.num_cores * info.num_subcores
    tw = n // nw
    mesh = plsc.VectorSubcoreMesh(core_axis_name="c", subcore_axis_name="s")

    @functools.partial(
        pl.kernel,
        out_type=jax.ShapeDtypeStruct((r_cap, c2), jnp.int32),
        mesh=mesh,
        scratch_types=[
            pltpu.VMEM((tw,), jnp.int32),
            pltpu.VMEM((tw,), jnp.int32),
            pltpu.VMEM((tw, c2), jnp.int32),
            pltpu.SemaphoreType.DMA,
            pltpu.SemaphoreType.DMA,
        ],
    )
    def scat(x_hbm, p0_hbm, p1_hbm, xs_hbm, idx0_v, idx1_v, rows_v, sem0, sem1):
        wid = lax.axis_index("s") * info.num_cores + lax.axis_index("c")
        base = wid * tw
        pltpu.sync_copy(p0_hbm.at[pl.ds(base, tw)], idx0_v)
        pltpu.sync_copy(p1_hbm.at[pl.ds(base, tw)], idx1_v)
        pltpu.sync_copy(x_hbm.at[pl.ds(base, tw)], rows_v)
        c0 = pltpu.async_copy(rows_v, xs_hbm.at[idx0_v], sem0)
        c1 = pltpu.async_copy(rows_v, xs_hbm.at[idx1_v], sem1)
        c0.wait()
        c1.wait()

    return scat(x_i32, pos0, pos1)


def _sc_gather_rows(ys_i32, pos0, pos1):
    n = pos0.shape[0]
    c2 = ys_i32.shape[1]
    info = plsc.get_sparse_core_info()
    nw = info.num_cores * info.num_subcores
    tw = n // nw
    mesh = plsc.VectorSubcoreMesh(core_axis_name="c", subcore_axis_name="s")

    @functools.partial(
        pl.kernel,
        out_type=(jax.ShapeDtypeStruct((n, c2), jnp.int32),
                  jax.ShapeDtypeStruct((n, c2), jnp.int32)),
        mesh=mesh,
        scratch_types=[
            pltpu.VMEM((tw,), jnp.int32),
            pltpu.VMEM((tw, c2), jnp.int32),
            pltpu.SemaphoreType.DMA,
        ],
    )
    def gath(ys_hbm, p0_hbm, p1_hbm, g0_hbm, g1_hbm, idx_v, rows_v, sem):
        wid = lax.axis_index("s") * info.num_cores + lax.axis_index("c")
        base = wid * tw
        pltpu.sync_copy(p0_hbm.at[pl.ds(base, tw)], idx_v)
        pltpu.async_copy(ys_hbm.at[idx_v], rows_v, sem).wait()
        pltpu.sync_copy(rows_v, g0_hbm.at[pl.ds(base, tw)])
        pltpu.sync_copy(p1_hbm.at[pl.ds(base, tw)], idx_v)
        pltpu.async_copy(ys_hbm.at[idx_v], rows_v, sem).wait()
        pltpu.sync_copy(rows_v, g1_hbm.at[pl.ds(base, tw)])

    return gath(ys_i32, pos0, pos1)


def _bf16_to_i32(a):
    n, c = a.shape
    return lax.bitcast_convert_type(a.reshape(n, c // 2, 2), jnp.int32)


def _i32_to_bf16(a):
    n, c2 = a.shape
    return lax.bitcast_convert_type(a, jnp.bfloat16).reshape(n, 2 * c2)


def _group_kernel(te_ref, tot_ref, xs_ref, wug_ref, wd_ref, ys_ref):
    t = pl.program_id(0)

    @pl.when(t * M_G < tot_ref[0])
    def _():
        ug = lax.dot_general(xs_ref[...], wug_ref[0], _NT,
                             preferred_element_type=jnp.float32)
        u = ug[:, :D_EXPERT]
        g = ug[:, D_EXPERT:]
        a = (u * lax.logistic(u) * g).astype(jnp.bfloat16)
        ys_ref[...] = lax.dot_general(a, wd_ref[0], _NT,
                                      preferred_element_type=jnp.float32
                                      ).astype(jnp.bfloat16)


def _combine_kernel(sh_ref, g0_ref, g1_ref, w2_ref, out_ref):
    w0 = w2_ref[:, 0:1]
    w1 = w2_ref[:, 1:2]
    out_ref[...] = (sh_ref[...] + w0 * g0_ref[...].astype(jnp.float32)
                    + w1 * g1_ref[...].astype(jnp.float32))


def kernel(x, shared_Wup, shared_Wgate, shared_Wdown,
           routed_Wup, routed_Wgate, routed_Wdown, router_W):
    B, T, C = x.shape
    N = B * T
    H = D_EXPERT
    x_flat = x.reshape(N, C)
    r_cap = TOP_K * N + N_ROUTED * M_G
    n_tiles = r_cap // M_G

    wug_sh = jnp.concatenate([shared_Wup, shared_Wgate], 0).astype(jnp.bfloat16)
    wd_sh = shared_Wdown.astype(jnp.bfloat16)
    wug_rt = jnp.concatenate([routed_Wup, routed_Wgate], 1).astype(jnp.bfloat16)
    wd_rt = routed_Wdown.astype(jnp.bfloat16)
    x_bf = x_flat.astype(jnp.bfloat16)

    m_tile = min(M_TILE, N)
    n_m = N // m_tile

    pos0, pos1, w2, te, tot = pl.pallas_call(
        functools.partial(_build_kernel, n_tokens=N, n_tiles=n_tiles),
        out_shape=[
            jax.ShapeDtypeStruct((N, 1), jnp.int32),
            jax.ShapeDtypeStruct((N, 1), jnp.int32),
            jax.ShapeDtypeStruct((N, TOP_K), jnp.float32),
            jax.ShapeDtypeStruct((n_tiles, 1), jnp.int32),
            jax.ShapeDtypeStruct((1, 1), jnp.int32),
        ],
    )(x_flat, router_W)

    pos0 = pos0.reshape(N)
    pos1 = pos1.reshape(N)

    xs = _i32_to_bf16(_sc_scatter_rows(_bf16_to_i32(x_bf), pos0, pos1, r_cap))

    shared_out = pl.pallas_call(
        _shared_kernel,
        grid=(n_m,),
        in_specs=[
            pl.BlockSpec((m_tile, C), lambda m: (m, 0)),
            pl.BlockSpec((2 * H, C), lambda m: (0, 0)),
            pl.BlockSpec((C, H), lambda m: (0, 0)),
        ],
        out_specs=pl.BlockSpec((m_tile, C), lambda m: (m, 0)),
        out_shape=jax.ShapeDtypeStruct((N, C), jnp.float32),
        compiler_params=pltpu.CompilerParams(
            dimension_semantics=("arbitrary",)),
    )(x_flat, wug_sh, wd_sh)

    grid_spec = pltpu.PrefetchScalarGridSpec(
        num_scalar_prefetch=2,
        grid=(n_tiles,),
        in_specs=[
            pl.BlockSpec((M_G, C), lambda t, te, tot: (t, 0)),
            pl.BlockSpec((1, 2 * H, C), lambda t, te, tot: (te[t], 0, 0)),
            pl.BlockSpec((1, C, H), lambda t, te, tot: (te[t], 0, 0)),
        ],
        out_specs=pl.BlockSpec((M_G, C), lambda t, te, tot: (t, 0)),
    )
    ys = pl.pallas_call(
        _group_kernel,
        grid_spec=grid_spec,
        out_shape=jax.ShapeDtypeStruct((r_cap, C), jnp.bfloat16),
        compiler_params=pltpu.CompilerParams(
            dimension_semantics=("arbitrary",)),
    )(te.reshape(n_tiles), tot.reshape(1), xs, wug_rt, wd_rt)

    g0i, g1i = _sc_gather_rows(_bf16_to_i32(ys), pos0, pos1)

    out = pl.pallas_call(
        _combine_kernel,
        grid=(n_m,),
        in_specs=[
            pl.BlockSpec((m_tile, C), lambda m: (m, 0)),
            pl.BlockSpec((m_tile, C), lambda m: (m, 0)),
            pl.BlockSpec((m_tile, C), lambda m: (m, 0)),
            pl.BlockSpec((m_tile, TOP_K), lambda m: (m, 0)),
        ],
        out_specs=pl.BlockSpec((m_tile, C), lambda m: (m, 0)),
        out_shape=jax.ShapeDtypeStruct((N, C), jnp.float32),
    )(shared_out, _i32_to_bf16(g0i), _i32_to_bf16(g1i), w2)

    return out.reshape(B, T, C)


# speedup vs baseline: 1.8374x; 1.8374x over previous
import functools

import jax
import jax.numpy as jnp
from jax import lax
from jax.experimental import pallas as pl
from jax.experimental.pallas import tpu as pltpu
from jax.experimental.pallas import tpu_sc as plsc

D_MODEL = 1024
D_EXPERT = 2048
N_ROUTED = 8
TOP_K = 2
M_G = 128
M_TILE = 512

_NT = (((1,), (1,)), ((), ()))


def _shared_kernel(x_ref, wug_ref, wd_ref, out_ref):
    xb = x_ref[...].astype(jnp.bfloat16)
    ug = lax.dot_general(xb, wug_ref[...], _NT, preferred_element_type=jnp.float32)
    u = ug[:, :D_EXPERT]
    g = ug[:, D_EXPERT:]
    a = (u * lax.logistic(u) * g).astype(jnp.bfloat16)
    out_ref[...] = lax.dot_general(a, wd_ref[...], _NT,
                                   preferred_element_type=jnp.float32)


def _build_kernel(x_ref, rw_ref, pos0_ref, pos1_ref, w2_ref, te_ref, tot_ref, *,
                  n_tokens, n_tiles):
    lg = lax.dot_general(x_ref[...], rw_ref[...], _NT,
                         preferred_element_type=jnp.float32)
    mx = jnp.max(lg, axis=-1, keepdims=True)
    p = jnp.exp(lg - mx)
    p = p / jnp.sum(p, axis=-1, keepdims=True)
    cols = lax.broadcasted_iota(jnp.int32, p.shape, 1)
    m1 = jnp.max(p, axis=-1, keepdims=True)
    i1 = jnp.min(jnp.where(p == m1, cols, N_ROUTED), axis=-1, keepdims=True)
    sel1 = cols == i1
    pm = jnp.where(sel1, -jnp.inf, p)
    m2 = jnp.max(pm, axis=-1, keepdims=True)
    i2 = jnp.min(jnp.where(pm == m2, cols, N_ROUTED), axis=-1, keepdims=True)
    den = m1 + m2 + 1e-8
    w2_ref[...] = jnp.concatenate([m1 / den, m2 / den], axis=1)

    key = jnp.concatenate([i1, i2], axis=0)
    oh = (key == lax.broadcasted_iota(jnp.int32, (2 * n_tokens, N_ROUTED), 1)
          ).astype(jnp.float32)
    inc = oh
    s = 1
    while s < 2 * n_tokens:
        inc = inc + jnp.concatenate(
            [jnp.zeros((s, N_ROUTED), jnp.float32), inc[:-s, :]], axis=0)
        s *= 2
    counts = inc[2 * n_tokens - 1:, :]
    cnt_pad = jnp.floor((counts + (M_G - 1)) * (1.0 / M_G)) * M_G
    incl = cnt_pad
    s = 1
    while s < N_ROUTED:
        incl = incl + jnp.concatenate(
            [jnp.zeros((1, s), jnp.float32), incl[:, :-s]], axis=1)
        s *= 2
    offs = incl - cnt_pad
    rank = jnp.sum(inc * oh, axis=1, keepdims=True) - 1.0
    offsel = jnp.sum(jnp.where(oh > 0.0, offs, 0.0), axis=1, keepdims=True)
    pos = (offsel + rank).astype(jnp.int32)
    pos0_ref[...] = pos[:n_tokens]
    pos1_ref[...] = pos[n_tokens:]

    t_row = (lax.broadcasted_iota(jnp.int32, (n_tiles, N_ROUTED), 0)
             * M_G).astype(jnp.float32)
    te = jnp.sum((offs <= t_row).astype(jnp.int32), axis=1, keepdims=True) - 1
    te_ref[...] = te
    tot_ref[...] = jnp.sum(cnt_pad, axis=1, keepdims=True).astype(jnp.int32)


def _sc_scatter_rows(x_flat, pos0, pos1, r_cap):
    n, c2 = x_flat.shape
    info = plsc.get_sparse_core_info()
    nw = info.num_cores * info.num_subcores
    tw = n // nw
    mesh = plsc.VectorSubcoreMesh(core_axis_name="c", subcore_axis_name="s")

    @functools.partial(
        pl.kernel,
        out_type=jax.ShapeDtypeStruct((r_cap, c2), jnp.float32),
        mesh=mesh,
        scratch_types=[
            pltpu.VMEM((tw,), jnp.int32),
            pltpu.VMEM((tw,), jnp.int32),
            pltpu.VMEM((tw, c2), jnp.float32),
            pltpu.SemaphoreType.DMA,
            pltpu.SemaphoreType.DMA,
        ],
    )
    def scat(x_hbm, p0_hbm, p1_hbm, xs_hbm, idx0_v, idx1_v, rows_v, sem0, sem1):
        wid = lax.axis_index("s") * info.num_cores + lax.axis_index("c")
        base = wid * tw
        pltpu.sync_copy(p0_hbm.at[pl.ds(base, tw)], idx0_v)
        pltpu.sync_copy(p1_hbm.at[pl.ds(base, tw)], idx1_v)
        pltpu.sync_copy(x_hbm.at[pl.ds(base, tw)], rows_v)
        c0 = pltpu.async_copy(rows_v, xs_hbm.at[idx0_v], sem0)
        c1 = pltpu.async_copy(rows_v, xs_hbm.at[idx1_v], sem1)
        c0.wait()
        c1.wait()

    return scat(x_flat, pos0, pos1)


def _sc_gather_rows(ys, pos0, pos1):
    n = pos0.shape[0]
    c2 = ys.shape[1]
    info = plsc.get_sparse_core_info()
    nw = info.num_cores * info.num_subcores
    tw = n // nw
    mesh = plsc.VectorSubcoreMesh(core_axis_name="c", subcore_axis_name="s")

    @functools.partial(
        pl.kernel,
        out_type=(jax.ShapeDtypeStruct((n, c2), jnp.float32),
                  jax.ShapeDtypeStruct((n, c2), jnp.float32)),
        mesh=mesh,
        scratch_types=[
            pltpu.VMEM((tw,), jnp.int32),
            pltpu.VMEM((tw, c2), jnp.float32),
            pltpu.SemaphoreType.DMA,
        ],
    )
    def gath(ys_hbm, p0_hbm, p1_hbm, g0_hbm, g1_hbm, idx_v, rows_v, sem):
        wid = lax.axis_index("s") * info.num_cores + lax.axis_index("c")
        base = wid * tw
        pltpu.sync_copy(p0_hbm.at[pl.ds(base, tw)], idx_v)
        pltpu.async_copy(ys_hbm.at[idx_v], rows_v, sem).wait()
        pltpu.sync_copy(rows_v, g0_hbm.at[pl.ds(base, tw)])
        pltpu.sync_copy(p1_hbm.at[pl.ds(base, tw)], idx_v)
        pltpu.async_copy(ys_hbm.at[idx_v], rows_v, sem).wait()
        pltpu.sync_copy(rows_v, g1_hbm.at[pl.ds(base, tw)])

    return gath(ys, pos0, pos1)


def _group_kernel(te_ref, tot_ref, xs_ref, wug_ref, wd_ref, ys_ref):
    t = pl.program_id(0)

    @pl.when(t * M_G < tot_ref[0])
    def _():
        ug = lax.dot_general(xs_ref[...].astype(jnp.bfloat16), wug_ref[0], _NT,
                             preferred_element_type=jnp.float32)
        u = ug[:, :D_EXPERT]
        g = ug[:, D_EXPERT:]
        a = (u * lax.logistic(u) * g).astype(jnp.bfloat16)
        ys_ref[...] = lax.dot_general(a, wd_ref[0], _NT,
                                      preferred_element_type=jnp.float32)


def _combine_kernel(sh_ref, g0_ref, g1_ref, w2_ref, out_ref):
    w0 = w2_ref[:, 0:1]
    w1 = w2_ref[:, 1:2]
    out_ref[...] = sh_ref[...] + w0 * g0_ref[...] + w1 * g1_ref[...]


def kernel(x, shared_Wup, shared_Wgate, shared_Wdown,
           routed_Wup, routed_Wgate, routed_Wdown, router_W):
    B, T, C = x.shape
    N = B * T
    H = D_EXPERT
    x_flat = x.reshape(N, C)
    r_cap = TOP_K * N + N_ROUTED * M_G
    n_tiles = r_cap // M_G

    wug_sh = jnp.concatenate([shared_Wup, shared_Wgate], 0).astype(jnp.bfloat16)
    wd_sh = shared_Wdown.astype(jnp.bfloat16)
    wug_rt = jnp.concatenate([routed_Wup, routed_Wgate], 1).astype(jnp.bfloat16)
    wd_rt = routed_Wdown.astype(jnp.bfloat16)

    m_tile = min(M_TILE, N)
    n_m = N // m_tile

    pos0, pos1, w2, te, tot = pl.pallas_call(
        functools.partial(_build_kernel, n_tokens=N, n_tiles=n_tiles),
        out_shape=[
            jax.ShapeDtypeStruct((N, 1), jnp.int32),
            jax.ShapeDtypeStruct((N, 1), jnp.int32),
            jax.ShapeDtypeStruct((N, TOP_K), jnp.float32),
            jax.ShapeDtypeStruct((n_tiles, 1), jnp.int32),
            jax.ShapeDtypeStruct((1, 1), jnp.int32),
        ],
    )(x_flat, router_W)

    pos0 = pos0.reshape(N)
    pos1 = pos1.reshape(N)

    xs = _sc_scatter_rows(x_flat, pos0, pos1, r_cap)

    shared_out = pl.pallas_call(
        _shared_kernel,
        grid=(n_m,),
        in_specs=[
            pl.BlockSpec((m_tile, C), lambda m: (m, 0)),
            pl.BlockSpec((2 * H, C), lambda m: (0, 0)),
            pl.BlockSpec((C, H), lambda m: (0, 0)),
        ],
        out_specs=pl.BlockSpec((m_tile, C), lambda m: (m, 0)),
        out_shape=jax.ShapeDtypeStruct((N, C), jnp.float32),
        compiler_params=pltpu.CompilerParams(
            dimension_semantics=("arbitrary",)),
    )(x_flat, wug_sh, wd_sh)

    grid_spec = pltpu.PrefetchScalarGridSpec(
        num_scalar_prefetch=2,
        grid=(n_tiles,),
        in_specs=[
            pl.BlockSpec((M_G, C), lambda t, te, tot: (t, 0)),
            pl.BlockSpec((1, 2 * H, C), lambda t, te, tot: (te[t], 0, 0)),
            pl.BlockSpec((1, C, H), lambda t, te, tot: (te[t], 0, 0)),
        ],
        out_specs=pl.BlockSpec((M_G, C), lambda t, te, tot: (t, 0)),
    )
    ys = pl.pallas_call(
        _group_kernel,
        grid_spec=grid_spec,
        out_shape=jax.ShapeDtypeStruct((r_cap, C), jnp.float32),
        compiler_params=pltpu.CompilerParams(
            dimension_semantics=("arbitrary",)),
    )(te.reshape(n_tiles), tot.reshape(1), xs, wug_rt, wd_rt)

    g0, g1 = _sc_gather_rows(ys, pos0, pos1)

    out = pl.pallas_call(
        _combine_kernel,
        grid=(n_m,),
        in_specs=[
            pl.BlockSpec((m_tile, C), lambda m: (m, 0)),
            pl.BlockSpec((m_tile, C), lambda m: (m, 0)),
            pl.BlockSpec((m_tile, C), lambda m: (m, 0)),
            pl.BlockSpec((m_tile, TOP_K), lambda m: (m, 0)),
        ],
        out_specs=pl.BlockSpec((m_tile, C), lambda m: (m, 0)),
        out_shape=jax.ShapeDtypeStruct((N, C), jnp.float32),
    )(shared_out, g0, g1, w2)

    return out.reshape(B, T, C)


# speedup vs baseline: 2.2050x; 1.2000x over previous
import functools

import jax
import jax.numpy as jnp
from jax import lax
from jax.experimental import pallas as pl
from jax.experimental.pallas import tpu as pltpu
from jax.experimental.pallas import tpu_sc as plsc

D_MODEL = 1024
D_EXPERT = 2048
N_ROUTED = 8
TOP_K = 2
M_G = 512
M_TILE = 512

_NT = (((1,), (1,)), ((), ()))


def _shared_kernel(x_ref, wug_ref, wd_ref, rw_ref, out_ref, logit_ref):
    x = x_ref[...]
    xb = x.astype(jnp.bfloat16)
    ug = lax.dot_general(xb, wug_ref[...], _NT, preferred_element_type=jnp.float32)
    u = ug[:, :D_EXPERT]
    g = ug[:, D_EXPERT:]
    a = (u * lax.logistic(u) * g).astype(jnp.bfloat16)
    out_ref[...] = lax.dot_general(a, wd_ref[...], _NT,
                                   preferred_element_type=jnp.float32)
    logit_ref[...] = lax.dot_general(x, rw_ref[...], _NT,
                                     preferred_element_type=jnp.float32)


def _build_kernel(logit_ref, pos0_ref, pos1_ref, w2_ref, te_ref, tot_ref, *,
                  n_tokens, n_tiles):
    lg = logit_ref[...]
    mx = jnp.max(lg, axis=-1, keepdims=True)
    p = jnp.exp(lg - mx)
    p = p / jnp.sum(p, axis=-1, keepdims=True)
    cols = lax.broadcasted_iota(jnp.int32, p.shape, 1)
    m1 = jnp.max(p, axis=-1, keepdims=True)
    i1 = jnp.min(jnp.where(p == m1, cols, N_ROUTED), axis=-1, keepdims=True)
    sel1 = cols == i1
    pm = jnp.where(sel1, -jnp.inf, p)
    m2 = jnp.max(pm, axis=-1, keepdims=True)
    i2 = jnp.min(jnp.where(pm == m2, cols, N_ROUTED), axis=-1, keepdims=True)
    den = m1 + m2 + 1e-8
    w2_ref[...] = jnp.concatenate([m1 / den, m2 / den], axis=1)

    key = jnp.concatenate([i1, i2], axis=0)
    oh = (key == lax.broadcasted_iota(jnp.int32, (2 * n_tokens, N_ROUTED), 1)
          ).astype(jnp.float32)
    inc = oh
    s = 1
    while s < 2 * n_tokens:
        inc = inc + jnp.concatenate(
            [jnp.zeros((s, N_ROUTED), jnp.float32), inc[:-s, :]], axis=0)
        s *= 2
    counts = inc[2 * n_tokens - 1:, :]
    cnt_pad = jnp.floor((counts + (M_G - 1)) * (1.0 / M_G)) * M_G
    incl = cnt_pad
    s = 1
    while s < N_ROUTED:
        incl = incl + jnp.concatenate(
            [jnp.zeros((1, s), jnp.float32), incl[:, :-s]], axis=1)
        s *= 2
    offs = incl - cnt_pad
    rank = jnp.sum(inc * oh, axis=1, keepdims=True) - 1.0
    offsel = jnp.sum(jnp.where(oh > 0.0, offs, 0.0), axis=1, keepdims=True)
    pos = (offsel + rank).astype(jnp.int32)
    pos0_ref[...] = pos[:n_tokens]
    pos1_ref[...] = pos[n_tokens:]

    t_row = (lax.broadcasted_iota(jnp.int32, (n_tiles, N_ROUTED), 0)
             * M_G).astype(jnp.float32)
    te = jnp.sum((offs <= t_row).astype(jnp.int32), axis=1, keepdims=True) - 1
    te_ref[...] = te
    tot_ref[...] = jnp.sum(cnt_pad, axis=1, keepdims=True).astype(jnp.int32)


def _sc_scatter_rows(x_flat, pos0, pos1, r_cap):
    n, c2 = x_flat.shape
    info = plsc.get_sparse_core_info()
    nw = info.num_cores * info.num_subcores
    tw = n // nw
    mesh = plsc.VectorSubcoreMesh(core_axis_name="c", subcore_axis_name="s")

    @functools.partial(
        pl.kernel,
        out_type=jax.ShapeDtypeStruct((r_cap, c2), jnp.float32),
        mesh=mesh,
        scratch_types=[
            pltpu.VMEM((tw,), jnp.int32),
            pltpu.VMEM((tw,), jnp.int32),
            pltpu.VMEM((tw, c2), jnp.float32),
            pltpu.SemaphoreType.DMA,
            pltpu.SemaphoreType.DMA,
        ],
    )
    def scat(x_hbm, p0_hbm, p1_hbm, xs_hbm, idx0_v, idx1_v, rows_v, sem0, sem1):
        wid = lax.axis_index("s") * info.num_cores + lax.axis_index("c")
        base = wid * tw
        pltpu.sync_copy(p0_hbm.at[pl.ds(base, tw)], idx0_v)
        pltpu.sync_copy(p1_hbm.at[pl.ds(base, tw)], idx1_v)
        pltpu.sync_copy(x_hbm.at[pl.ds(base, tw)], rows_v)
        c0 = pltpu.async_copy(rows_v, xs_hbm.at[idx0_v], sem0)
        c1 = pltpu.async_copy(rows_v, xs_hbm.at[idx1_v], sem1)
        c0.wait()
        c1.wait()

    return scat(x_flat, pos0, pos1)


def _sc_gather_rows(ys, pos0, pos1):
    n = pos0.shape[0]
    c2 = ys.shape[1]
    info = plsc.get_sparse_core_info()
    nw = info.num_cores * info.num_subcores
    tw = n // nw
    mesh = plsc.VectorSubcoreMesh(core_axis_name="c", subcore_axis_name="s")

    @functools.partial(
        pl.kernel,
        out_type=(jax.ShapeDtypeStruct((n, c2), jnp.float32),
                  jax.ShapeDtypeStruct((n, c2), jnp.float32)),
        mesh=mesh,
        scratch_types=[
            pltpu.VMEM((tw,), jnp.int32),
            pltpu.VMEM((tw, c2), jnp.float32),
            pltpu.SemaphoreType.DMA,
        ],
    )
    def gath(ys_hbm, p0_hbm, p1_hbm, g0_hbm, g1_hbm, idx_v, rows_v, sem):
        wid = lax.axis_index("s") * info.num_cores + lax.axis_index("c")
        base = wid * tw
        pltpu.sync_copy(p0_hbm.at[pl.ds(base, tw)], idx_v)
        pltpu.async_copy(ys_hbm.at[idx_v], rows_v, sem).wait()
        pltpu.sync_copy(rows_v, g0_hbm.at[pl.ds(base, tw)])
        pltpu.sync_copy(p1_hbm.at[pl.ds(base, tw)], idx_v)
        pltpu.async_copy(ys_hbm.at[idx_v], rows_v, sem).wait()
        pltpu.sync_copy(rows_v, g1_hbm.at[pl.ds(base, tw)])

    return gath(ys, pos0, pos1)


def _group_kernel(te_ref, tot_ref, xs_ref, wug_ref, wd_ref, ys_ref):
    t = pl.program_id(0)

    @pl.when(t * M_G < tot_ref[0])
    def _():
        ug = lax.dot_general(xs_ref[...].astype(jnp.bfloat16), wug_ref[0], _NT,
                             preferred_element_type=jnp.float32)
        u = ug[:, :D_EXPERT]
        g = ug[:, D_EXPERT:]
        a = (u * lax.logistic(u) * g).astype(jnp.bfloat16)
        ys_ref[...] = lax.dot_general(a, wd_ref[0], _NT,
                                      preferred_element_type=jnp.float32)


def _combine_kernel(sh_ref, g0_ref, g1_ref, w2_ref, out_ref):
    w0 = w2_ref[:, 0:1]
    w1 = w2_ref[:, 1:2]
    out_ref[...] = sh_ref[...] + w0 * g0_ref[...] + w1 * g1_ref[...]


def kernel(x, shared_Wup, shared_Wgate, shared_Wdown,
           routed_Wup, routed_Wgate, routed_Wdown, router_W):
    B, T, C = x.shape
    N = B * T
    H = D_EXPERT
    x_flat = x.reshape(N, C)
    r_cap = TOP_K * N + N_ROUTED * M_G
    n_tiles = r_cap // M_G

    wug_sh = jnp.concatenate([shared_Wup, shared_Wgate], 0).astype(jnp.bfloat16)
    wd_sh = shared_Wdown.astype(jnp.bfloat16)
    wug_rt = jnp.concatenate([routed_Wup, routed_Wgate], 1).astype(jnp.bfloat16)
    wd_rt = routed_Wdown.astype(jnp.bfloat16)

    m_tile = min(M_TILE, N)
    n_m = N // m_tile

    shared_out, logits = pl.pallas_call(
        _shared_kernel,
        grid=(n_m,),
        in_specs=[
            pl.BlockSpec((m_tile, C), lambda m: (m, 0)),
            pl.BlockSpec((2 * H, C), lambda m: (0, 0)),
            pl.BlockSpec((C, H), lambda m: (0, 0)),
            pl.BlockSpec((N_ROUTED, C), lambda m: (0, 0)),
        ],
        out_specs=[
            pl.BlockSpec((m_tile, C), lambda m: (m, 0)),
            pl.BlockSpec((m_tile, N_ROUTED), lambda m: (m, 0)),
        ],
        out_shape=[
            jax.ShapeDtypeStruct((N, C), jnp.float32),
            jax.ShapeDtypeStruct((N, N_ROUTED), jnp.float32),
        ],
        compiler_params=pltpu.CompilerParams(
            dimension_semantics=("arbitrary",)),
    )(x_flat, wug_sh, wd_sh, router_W)

    pos0, pos1, w2, te, tot = pl.pallas_call(
        functools.partial(_build_kernel, n_tokens=N, n_tiles=n_tiles),
        out_shape=[
            jax.ShapeDtypeStruct((N, 1), jnp.int32),
            jax.ShapeDtypeStruct((N, 1), jnp.int32),
            jax.ShapeDtypeStruct((N, TOP_K), jnp.float32),
            jax.ShapeDtypeStruct((n_tiles, 1), jnp.int32),
            jax.ShapeDtypeStruct((1, 1), jnp.int32),
        ],
    )(logits)

    pos0 = pos0.reshape(N)
    pos1 = pos1.reshape(N)

    xs = _sc_scatter_rows(x_flat, pos0, pos1, r_cap)

    grid_spec = pltpu.PrefetchScalarGridSpec(
        num_scalar_prefetch=2,
        grid=(n_tiles,),
        in_specs=[
            pl.BlockSpec((M_G, C), lambda t, te, tot: (t, 0)),
            pl.BlockSpec((1, 2 * H, C), lambda t, te, tot: (te[t], 0, 0)),
            pl.BlockSpec((1, C, H), lambda t, te, tot: (te[t], 0, 0)),
        ],
        out_specs=pl.BlockSpec((M_G, C), lambda t, te, tot: (t, 0)),
    )
    ys = pl.pallas_call(
        _group_kernel,
        grid_spec=grid_spec,
        out_shape=jax.ShapeDtypeStruct((r_cap, C), jnp.float32),
        compiler_params=pltpu.CompilerParams(
            dimension_semantics=("arbitrary",)),
    )(te.reshape(n_tiles), tot.reshape(1), xs, wug_rt, wd_rt)

    g0, g1 = _sc_gather_rows(ys, pos0, pos1)

    out = pl.pallas_call(
        _combine_kernel,
        grid=(n_m,),
        in_specs=[
            pl.BlockSpec((m_tile, C), lambda m: (m, 0)),
            pl.BlockSpec((m_tile, C), lambda m: (m, 0)),
            pl.BlockSpec((m_tile, C), lambda m: (m, 0)),
            pl.BlockSpec((m_tile, TOP_K), lambda m: (m, 0)),
        ],
        out_specs=pl.BlockSpec((m_tile, C), lambda m: (m, 0)),
        out_shape=jax.ShapeDtypeStruct((N, C), jnp.float32),
    )(shared_out, g0, g1, w2)

    return out.reshape(B, T, C)


# speedup vs baseline: 3.1808x; 1.4426x over previous
import functools

import jax
import jax.numpy as jnp
from jax import lax
from jax.experimental import pallas as pl
from jax.experimental.pallas import tpu as pltpu
from jax.experimental.pallas import tpu_sc as plsc

D_MODEL = 1024
D_EXPERT = 2048
N_ROUTED = 8
TOP_K = 2
M_G = 512
M_TILE = 512

_NT = (((1,), (1,)), ((), ()))


def _shared_kernel(x_ref, wu_ref, wg_ref, wd_ref, rw_ref, out_ref, logit_ref,
                   wub_ref, wgb_ref, wdb_ref):
    m = pl.program_id(0)

    @pl.when(m == 0)
    def _():
        wub_ref[...] = wu_ref[...].astype(jnp.bfloat16)
        wgb_ref[...] = wg_ref[...].astype(jnp.bfloat16)
        wdb_ref[...] = wd_ref[...].astype(jnp.bfloat16)

    x = x_ref[...]
    xb = x.astype(jnp.bfloat16)
    u = lax.dot_general(xb, wub_ref[...], _NT, preferred_element_type=jnp.float32)
    g = lax.dot_general(xb, wgb_ref[...], _NT, preferred_element_type=jnp.float32)
    a = (u * lax.logistic(u) * g).astype(jnp.bfloat16)
    out_ref[...] = lax.dot_general(a, wdb_ref[...], _NT,
                                   preferred_element_type=jnp.float32)
    logit_ref[...] = lax.dot_general(x, rw_ref[...], _NT,
                                     preferred_element_type=jnp.float32)


def _build_kernel(logit_ref, pos0_ref, pos1_ref, w2_ref, te_ref, tot_ref, *,
                  n_tokens, n_tiles):
    lg = logit_ref[...]
    mx = jnp.max(lg, axis=-1, keepdims=True)
    p = jnp.exp(lg - mx)
    p = p / jnp.sum(p, axis=-1, keepdims=True)
    cols = lax.broadcasted_iota(jnp.int32, p.shape, 1)
    m1 = jnp.max(p, axis=-1, keepdims=True)
    i1 = jnp.min(jnp.where(p == m1, cols, N_ROUTED), axis=-1, keepdims=True)
    sel1 = cols == i1
    pm = jnp.where(sel1, -jnp.inf, p)
    m2 = jnp.max(pm, axis=-1, keepdims=True)
    i2 = jnp.min(jnp.where(pm == m2, cols, N_ROUTED), axis=-1, keepdims=True)
    den = m1 + m2 + 1e-8
    w2_ref[...] = jnp.concatenate([m1 / den, m2 / den], axis=1)

    key = jnp.concatenate([i1, i2], axis=0)
    oh = (key == lax.broadcasted_iota(jnp.int32, (2 * n_tokens, N_ROUTED), 1)
          ).astype(jnp.float32)
    inc = oh
    s = 1
    while s < 2 * n_tokens:
        inc = inc + jnp.concatenate(
            [jnp.zeros((s, N_ROUTED), jnp.float32), inc[:-s, :]], axis=0)
        s *= 2
    counts = inc[2 * n_tokens - 1:, :]
    cnt_pad = jnp.floor((counts + (M_G - 1)) * (1.0 / M_G)) * M_G
    incl = cnt_pad
    s = 1
    while s < N_ROUTED:
        incl = incl + jnp.concatenate(
            [jnp.zeros((1, s), jnp.float32), incl[:, :-s]], axis=1)
        s *= 2
    offs = incl - cnt_pad
    rank = jnp.sum(inc * oh, axis=1, keepdims=True) - 1.0
    offsel = jnp.sum(jnp.where(oh > 0.0, offs, 0.0), axis=1, keepdims=True)
    pos = (offsel + rank).astype(jnp.int32)
    pos0_ref[...] = pos[:n_tokens]
    pos1_ref[...] = pos[n_tokens:]

    t_row = (lax.broadcasted_iota(jnp.int32, (n_tiles, N_ROUTED), 0)
             * M_G).astype(jnp.float32)
    te = jnp.sum((offs <= t_row).astype(jnp.int32), axis=1, keepdims=True) - 1
    te_ref[...] = te
    tot_ref[...] = jnp.sum(cnt_pad, axis=1, keepdims=True).astype(jnp.int32)


def _sc_scatter_rows(x_flat, pos0, pos1, r_cap):
    n, c2 = x_flat.shape
    info = plsc.get_sparse_core_info()
    nw = info.num_cores * info.num_subcores
    tw = n // nw
    mesh = plsc.VectorSubcoreMesh(core_axis_name="c", subcore_axis_name="s")

    @functools.partial(
        pl.kernel,
        out_type=jax.ShapeDtypeStruct((r_cap, c2), jnp.float32),
        mesh=mesh,
        scratch_types=[
            pltpu.VMEM((tw,), jnp.int32),
            pltpu.VMEM((tw,), jnp.int32),
            pltpu.VMEM((tw, c2), jnp.float32),
            pltpu.SemaphoreType.DMA,
            pltpu.SemaphoreType.DMA,
        ],
    )
    def scat(x_hbm, p0_hbm, p1_hbm, xs_hbm, idx0_v, idx1_v, rows_v, sem0, sem1):
        wid = lax.axis_index("s") * info.num_cores + lax.axis_index("c")
        base = wid * tw
        pltpu.sync_copy(p0_hbm.at[pl.ds(base, tw)], idx0_v)
        pltpu.sync_copy(p1_hbm.at[pl.ds(base, tw)], idx1_v)
        pltpu.sync_copy(x_hbm.at[pl.ds(base, tw)], rows_v)
        c0 = pltpu.async_copy(rows_v, xs_hbm.at[idx0_v], sem0)
        c1 = pltpu.async_copy(rows_v, xs_hbm.at[idx1_v], sem1)
        c0.wait()
        c1.wait()

    return scat(x_flat, pos0, pos1)


def _sc_gather_rows(ys, pos0, pos1):
    n = pos0.shape[0]
    c2 = ys.shape[1]
    info = plsc.get_sparse_core_info()
    nw = info.num_cores * info.num_subcores
    tw = n // nw
    mesh = plsc.VectorSubcoreMesh(core_axis_name="c", subcore_axis_name="s")

    @functools.partial(
        pl.kernel,
        out_type=(jax.ShapeDtypeStruct((n, c2), jnp.float32),
                  jax.ShapeDtypeStruct((n, c2), jnp.float32)),
        mesh=mesh,
        scratch_types=[
            pltpu.VMEM((tw,), jnp.int32),
            pltpu.VMEM((tw, c2), jnp.float32),
            pltpu.SemaphoreType.DMA,
        ],
    )
    def gath(ys_hbm, p0_hbm, p1_hbm, g0_hbm, g1_hbm, idx_v, rows_v, sem):
        wid = lax.axis_index("s") * info.num_cores + lax.axis_index("c")
        base = wid * tw
        pltpu.sync_copy(p0_hbm.at[pl.ds(base, tw)], idx_v)
        pltpu.async_copy(ys_hbm.at[idx_v], rows_v, sem).wait()
        pltpu.sync_copy(rows_v, g0_hbm.at[pl.ds(base, tw)])
        pltpu.sync_copy(p1_hbm.at[pl.ds(base, tw)], idx_v)
        pltpu.async_copy(ys_hbm.at[idx_v], rows_v, sem).wait()
        pltpu.sync_copy(rows_v, g1_hbm.at[pl.ds(base, tw)])

    return gath(ys, pos0, pos1)


def _group_kernel(te_ref, tot_ref, xs_ref, wu_ref, wg_ref, wd_ref, ys_ref,
                  xsb_ref, a_ref):
    t = pl.program_id(0)
    h = pl.program_id(1)
    hh = D_EXPERT // 2

    @pl.when(t * M_G < tot_ref[0])
    def _():
        @pl.when(h == 0)
        def _():
            xsb_ref[...] = xs_ref[...].astype(jnp.bfloat16)

        xb = xsb_ref[...]
        u = lax.dot_general(xb, wu_ref[0].astype(jnp.bfloat16), _NT,
                            preferred_element_type=jnp.float32)
        g = lax.dot_general(xb, wg_ref[0].astype(jnp.bfloat16), _NT,
                            preferred_element_type=jnp.float32)
        a_ref[:, pl.ds(h * hh, hh)] = (u * lax.logistic(u) * g).astype(jnp.bfloat16)

        @pl.when(h == 1)
        def _():
            ys_ref[...] = lax.dot_general(
                a_ref[...], wd_ref[0].astype(jnp.bfloat16), _NT,
                preferred_element_type=jnp.float32)


def _combine_kernel(sh_ref, g0_ref, g1_ref, w2_ref, out_ref):
    w0 = w2_ref[:, 0:1]
    w1 = w2_ref[:, 1:2]
    out_ref[...] = sh_ref[...] + w0 * g0_ref[...] + w1 * g1_ref[...]


def kernel(x, shared_Wup, shared_Wgate, shared_Wdown,
           routed_Wup, routed_Wgate, routed_Wdown, router_W):
    B, T, C = x.shape
    N = B * T
    H = D_EXPERT
    x_flat = x.reshape(N, C)
    r_cap = TOP_K * N + N_ROUTED * M_G
    n_tiles = r_cap // M_G


    m_tile = min(M_TILE, N)
    n_m = N // m_tile

    shared_out, logits = pl.pallas_call(
        _shared_kernel,
        grid=(n_m,),
        in_specs=[
            pl.BlockSpec((m_tile, C), lambda m: (m, 0)),
            pl.BlockSpec((H, C), lambda m: (0, 0)),
            pl.BlockSpec((H, C), lambda m: (0, 0)),
            pl.BlockSpec((C, H), lambda m: (0, 0)),
            pl.BlockSpec((N_ROUTED, C), lambda m: (0, 0)),
        ],
        out_specs=[
            pl.BlockSpec((m_tile, C), lambda m: (m, 0)),
            pl.BlockSpec((m_tile, N_ROUTED), lambda m: (m, 0)),
        ],
        out_shape=[
            jax.ShapeDtypeStruct((N, C), jnp.float32),
            jax.ShapeDtypeStruct((N, N_ROUTED), jnp.float32),
        ],
        scratch_shapes=[
            pltpu.VMEM((H, C), jnp.bfloat16),
            pltpu.VMEM((H, C), jnp.bfloat16),
            pltpu.VMEM((C, H), jnp.bfloat16),
        ],
        compiler_params=pltpu.CompilerParams(
            dimension_semantics=("arbitrary",)),
    )(x_flat, shared_Wup, shared_Wgate, shared_Wdown, router_W)

    pos0, pos1, w2, te, tot = pl.pallas_call(
        functools.partial(_build_kernel, n_tokens=N, n_tiles=n_tiles),
        out_shape=[
            jax.ShapeDtypeStruct((N, 1), jnp.int32),
            jax.ShapeDtypeStruct((N, 1), jnp.int32),
            jax.ShapeDtypeStruct((N, TOP_K), jnp.float32),
            jax.ShapeDtypeStruct((n_tiles, 1), jnp.int32),
            jax.ShapeDtypeStruct((1, 1), jnp.int32),
        ],
    )(logits)

    pos0 = pos0.reshape(N)
    pos1 = pos1.reshape(N)

    xs = _sc_scatter_rows(x_flat, pos0, pos1, r_cap)

    hh = H // 2
    grid_spec = pltpu.PrefetchScalarGridSpec(
        num_scalar_prefetch=2,
        grid=(n_tiles, 2),
        in_specs=[
            pl.BlockSpec((M_G, C), lambda t, h, te, tot: (t, 0)),
            pl.BlockSpec((1, hh, C), lambda t, h, te, tot: (te[t], h, 0)),
            pl.BlockSpec((1, hh, C), lambda t, h, te, tot: (te[t], h, 0)),
            pl.BlockSpec((1, C, H), lambda t, h, te, tot: (te[t], 0, 0)),
        ],
        out_specs=pl.BlockSpec((M_G, C), lambda t, h, te, tot: (t, 0)),
        scratch_shapes=[
            pltpu.VMEM((M_G, C), jnp.bfloat16),
            pltpu.VMEM((M_G, H), jnp.bfloat16),
        ],
    )
    ys = pl.pallas_call(
        _group_kernel,
        grid_spec=grid_spec,
        out_shape=jax.ShapeDtypeStruct((r_cap, C), jnp.float32),
        compiler_params=pltpu.CompilerParams(
            dimension_semantics=("arbitrary", "arbitrary")),
    )(te.reshape(n_tiles), tot.reshape(1), xs, routed_Wup, routed_Wgate,
      routed_Wdown)

    g0, g1 = _sc_gather_rows(ys, pos0, pos1)

    out = pl.pallas_call(
        _combine_kernel,
        grid=(n_m,),
        in_specs=[
            pl.BlockSpec((m_tile, C), lambda m: (m, 0)),
            pl.BlockSpec((m_tile, C), lambda m: (m, 0)),
            pl.BlockSpec((m_tile, C), lambda m: (m, 0)),
            pl.BlockSpec((m_tile, TOP_K), lambda m: (m, 0)),
        ],
        out_specs=pl.BlockSpec((m_tile, C), lambda m: (m, 0)),
        out_shape=jax.ShapeDtypeStruct((N, C), jnp.float32),
    )(shared_out, g0, g1, w2)

    return out.reshape(B, T, C)


# speedup vs baseline: 3.2421x; 1.0193x over previous
import functools

import jax
import jax.numpy as jnp
from jax import lax
from jax.experimental import pallas as pl
from jax.experimental.pallas import tpu as pltpu
from jax.experimental.pallas import tpu_sc as plsc

D_MODEL = 1024
D_EXPERT = 2048
N_ROUTED = 8
TOP_K = 2
M_G = 512
M_TILE = 512

_NT = (((1,), (1,)), ((), ()))


def _shared_kernel(x_ref, wu_ref, wg_ref, wd_ref, out_ref,
                   wub_ref, wgb_ref, wdb_ref):
    m = pl.program_id(0)

    @pl.when(m == 0)
    def _():
        wub_ref[...] = wu_ref[...].astype(jnp.bfloat16)
        wgb_ref[...] = wg_ref[...].astype(jnp.bfloat16)
        wdb_ref[...] = wd_ref[...].astype(jnp.bfloat16)

    x = x_ref[...]
    xb = x.astype(jnp.bfloat16)
    u = lax.dot_general(xb, wub_ref[...], _NT, preferred_element_type=jnp.float32)
    g = lax.dot_general(xb, wgb_ref[...], _NT, preferred_element_type=jnp.float32)
    a = (u * lax.logistic(u) * g).astype(jnp.bfloat16)
    out_ref[...] = lax.dot_general(a, wdb_ref[...], _NT,
                                   preferred_element_type=jnp.float32)


def _build_kernel(x_ref, rw_ref, pos0_ref, pos1_ref, w2_ref, te_ref, tot_ref, *,
                  n_tokens, n_tiles):
    lg = lax.dot_general(x_ref[...], rw_ref[...], _NT,
                         preferred_element_type=jnp.float32)
    mx = jnp.max(lg, axis=-1, keepdims=True)
    p = jnp.exp(lg - mx)
    p = p / jnp.sum(p, axis=-1, keepdims=True)
    cols = lax.broadcasted_iota(jnp.int32, p.shape, 1)
    m1 = jnp.max(p, axis=-1, keepdims=True)
    i1 = jnp.min(jnp.where(p == m1, cols, N_ROUTED), axis=-1, keepdims=True)
    sel1 = cols == i1
    pm = jnp.where(sel1, -jnp.inf, p)
    m2 = jnp.max(pm, axis=-1, keepdims=True)
    i2 = jnp.min(jnp.where(pm == m2, cols, N_ROUTED), axis=-1, keepdims=True)
    den = m1 + m2 + 1e-8
    w2_ref[...] = jnp.concatenate([m1 / den, m2 / den], axis=1)

    key = jnp.concatenate([i1, i2], axis=0)
    oh = (key == lax.broadcasted_iota(jnp.int32, (2 * n_tokens, N_ROUTED), 1)
          ).astype(jnp.float32)
    inc = oh
    s = 1
    while s < 2 * n_tokens:
        inc = inc + jnp.concatenate(
            [jnp.zeros((s, N_ROUTED), jnp.float32), inc[:-s, :]], axis=0)
        s *= 2
    counts = inc[2 * n_tokens - 1:, :]
    cnt_pad = jnp.floor((counts + (M_G - 1)) * (1.0 / M_G)) * M_G
    incl = cnt_pad
    s = 1
    while s < N_ROUTED:
        incl = incl + jnp.concatenate(
            [jnp.zeros((1, s), jnp.float32), incl[:, :-s]], axis=1)
        s *= 2
    offs = incl - cnt_pad
    rank = jnp.sum(inc * oh, axis=1, keepdims=True) - 1.0
    offsel = jnp.sum(jnp.where(oh > 0.0, offs, 0.0), axis=1, keepdims=True)
    pos = (offsel + rank).astype(jnp.int32)
    pos0_ref[...] = pos[:n_tokens]
    pos1_ref[...] = pos[n_tokens:]

    t_row = (lax.broadcasted_iota(jnp.int32, (n_tiles, N_ROUTED), 0)
             * M_G).astype(jnp.float32)
    te = jnp.sum((offs <= t_row).astype(jnp.int32), axis=1, keepdims=True) - 1
    te_ref[...] = te
    tot_ref[...] = jnp.sum(cnt_pad, axis=1, keepdims=True).astype(jnp.int32)


def _sc_scatter_rows(x_flat, pos0, pos1, r_cap):
    n, c2 = x_flat.shape
    info = plsc.get_sparse_core_info()
    nw = info.num_cores * info.num_subcores
    tw = n // nw
    mesh = plsc.VectorSubcoreMesh(core_axis_name="c", subcore_axis_name="s")

    @functools.partial(
        pl.kernel,
        out_type=jax.ShapeDtypeStruct((r_cap, c2), jnp.float32),
        mesh=mesh,
        scratch_types=[
            pltpu.VMEM((tw,), jnp.int32),
            pltpu.VMEM((tw,), jnp.int32),
            pltpu.VMEM((tw, c2), jnp.float32),
            pltpu.SemaphoreType.DMA,
            pltpu.SemaphoreType.DMA,
        ],
    )
    def scat(x_hbm, p0_hbm, p1_hbm, xs_hbm, idx0_v, idx1_v, rows_v, sem0, sem1):
        wid = lax.axis_index("s") * info.num_cores + lax.axis_index("c")
        base = wid * tw
        pltpu.sync_copy(p0_hbm.at[pl.ds(base, tw)], idx0_v)
        pltpu.sync_copy(p1_hbm.at[pl.ds(base, tw)], idx1_v)
        pltpu.sync_copy(x_hbm.at[pl.ds(base, tw)], rows_v)
        c0 = pltpu.async_copy(rows_v, xs_hbm.at[idx0_v], sem0)
        c1 = pltpu.async_copy(rows_v, xs_hbm.at[idx1_v], sem1)
        c0.wait()
        c1.wait()

    return scat(x_flat, pos0, pos1)


def _sc_gather_rows(ys, pos0, pos1):
    n = pos0.shape[0]
    c2 = ys.shape[1]
    info = plsc.get_sparse_core_info()
    nw = info.num_cores * info.num_subcores
    tw = n // nw
    mesh = plsc.VectorSubcoreMesh(core_axis_name="c", subcore_axis_name="s")

    @functools.partial(
        pl.kernel,
        out_type=(jax.ShapeDtypeStruct((n, c2), jnp.float32),
                  jax.ShapeDtypeStruct((n, c2), jnp.float32)),
        mesh=mesh,
        scratch_types=[
            pltpu.VMEM((tw,), jnp.int32),
            pltpu.VMEM((tw, c2), jnp.float32),
            pltpu.SemaphoreType.DMA,
        ],
    )
    def gath(ys_hbm, p0_hbm, p1_hbm, g0_hbm, g1_hbm, idx_v, rows_v, sem):
        wid = lax.axis_index("s") * info.num_cores + lax.axis_index("c")
        base = wid * tw
        pltpu.sync_copy(p0_hbm.at[pl.ds(base, tw)], idx_v)
        pltpu.async_copy(ys_hbm.at[idx_v], rows_v, sem).wait()
        pltpu.sync_copy(rows_v, g0_hbm.at[pl.ds(base, tw)])
        pltpu.sync_copy(p1_hbm.at[pl.ds(base, tw)], idx_v)
        pltpu.async_copy(ys_hbm.at[idx_v], rows_v, sem).wait()
        pltpu.sync_copy(rows_v, g1_hbm.at[pl.ds(base, tw)])

    return gath(ys, pos0, pos1)


def _group_kernel(te_ref, tot_ref, xs_ref, wu_ref, wg_ref, wd_ref, ys_ref,
                  xsb_ref, a_ref):
    t = pl.program_id(0)
    h = pl.program_id(1)
    hh = D_EXPERT // 2

    @pl.when(t * M_G < tot_ref[0])
    def _():
        @pl.when(h == 0)
        def _():
            xsb_ref[...] = xs_ref[...].astype(jnp.bfloat16)

        xb = xsb_ref[...]
        u = lax.dot_general(xb, wu_ref[0].astype(jnp.bfloat16), _NT,
                            preferred_element_type=jnp.float32)
        g = lax.dot_general(xb, wg_ref[0].astype(jnp.bfloat16), _NT,
                            preferred_element_type=jnp.float32)
        a_ref[:, pl.ds(h * hh, hh)] = (u * lax.logistic(u) * g).astype(jnp.bfloat16)

        @pl.when(h == 1)
        def _():
            ys_ref[...] = lax.dot_general(
                a_ref[...], wd_ref[0].astype(jnp.bfloat16), _NT,
                preferred_element_type=jnp.float32)


def _combine_kernel(sh_ref, g0_ref, g1_ref, w2_ref, out_ref):
    w0 = w2_ref[:, 0:1]
    w1 = w2_ref[:, 1:2]
    out_ref[...] = sh_ref[...] + w0 * g0_ref[...] + w1 * g1_ref[...]


def kernel(x, shared_Wup, shared_Wgate, shared_Wdown,
           routed_Wup, routed_Wgate, routed_Wdown, router_W):
    B, T, C = x.shape
    N = B * T
    H = D_EXPERT
    x_flat = x.reshape(N, C)
    r_cap = TOP_K * N + N_ROUTED * M_G
    n_tiles = r_cap // M_G


    m_tile = min(M_TILE, N)
    n_m = N // m_tile

    pos0, pos1, w2, te, tot = pl.pallas_call(
        functools.partial(_build_kernel, n_tokens=N, n_tiles=n_tiles),
        out_shape=[
            jax.ShapeDtypeStruct((N, 1), jnp.int32),
            jax.ShapeDtypeStruct((N, 1), jnp.int32),
            jax.ShapeDtypeStruct((N, TOP_K), jnp.float32),
            jax.ShapeDtypeStruct((n_tiles, 1), jnp.int32),
            jax.ShapeDtypeStruct((1, 1), jnp.int32),
        ],
    )(x_flat, router_W)

    pos0 = pos0.reshape(N)
    pos1 = pos1.reshape(N)

    xs = _sc_scatter_rows(x_flat, pos0, pos1, r_cap)

    shared_out = pl.pallas_call(
        _shared_kernel,
        grid=(n_m,),
        in_specs=[
            pl.BlockSpec((m_tile, C), lambda m: (m, 0)),
            pl.BlockSpec((H, C), lambda m: (0, 0)),
            pl.BlockSpec((H, C), lambda m: (0, 0)),
            pl.BlockSpec((C, H), lambda m: (0, 0)),
        ],
        out_specs=pl.BlockSpec((m_tile, C), lambda m: (m, 0)),
        out_shape=jax.ShapeDtypeStruct((N, C), jnp.float32),
        scratch_shapes=[
            pltpu.VMEM((H, C), jnp.bfloat16),
            pltpu.VMEM((H, C), jnp.bfloat16),
            pltpu.VMEM((C, H), jnp.bfloat16),
        ],
        compiler_params=pltpu.CompilerParams(
            dimension_semantics=("arbitrary",)),
    )(x_flat, shared_Wup, shared_Wgate, shared_Wdown)

    hh = H // 2
    grid_spec = pltpu.PrefetchScalarGridSpec(
        num_scalar_prefetch=2,
        grid=(n_tiles, 2),
        in_specs=[
            pl.BlockSpec((M_G, C), lambda t, h, te, tot: (t, 0)),
            pl.BlockSpec((1, hh, C), lambda t, h, te, tot: (te[t], h, 0)),
            pl.BlockSpec((1, hh, C), lambda t, h, te, tot: (te[t], h, 0)),
            pl.BlockSpec((1, C, H), lambda t, h, te, tot: (te[t], 0, 0)),
        ],
        out_specs=pl.BlockSpec((M_G, C), lambda t, h, te, tot: (t, 0)),
        scratch_shapes=[
            pltpu.VMEM((M_G, C), jnp.bfloat16),
            pltpu.VMEM((M_G, H), jnp.bfloat16),
        ],
    )
    ys = pl.pallas_call(
        _group_kernel,
        grid_spec=grid_spec,
        out_shape=jax.ShapeDtypeStruct((r_cap, C), jnp.float32),
        compiler_params=pltpu.CompilerParams(
            dimension_semantics=("arbitrary", "arbitrary")),
    )(te.reshape(n_tiles), tot.reshape(1), xs, routed_Wup, routed_Wgate,
      routed_Wdown)

    g0, g1 = _sc_gather_rows(ys, pos0, pos1)

    out = pl.pallas_call(
        _combine_kernel,
        grid=(n_m,),
        in_specs=[
            pl.BlockSpec((m_tile, C), lambda m: (m, 0)),
            pl.BlockSpec((m_tile, C), lambda m: (m, 0)),
            pl.BlockSpec((m_tile, C), lambda m: (m, 0)),
            pl.BlockSpec((m_tile, TOP_K), lambda m: (m, 0)),
        ],
        out_specs=pl.BlockSpec((m_tile, C), lambda m: (m, 0)),
        out_shape=jax.ShapeDtypeStruct((N, C), jnp.float32),
    )(shared_out, g0, g1, w2)

    return out.reshape(B, T, C)
